# baseline (speedup 1.0000x reference)
import jax
import jax.numpy as jnp
from jax import lax
from jax.experimental import pallas as pl
from jax.experimental.pallas import tpu as pltpu

_B = 1024
_C = 100000
_W = 4096
_NBLK = (_C + _W - 1) // _W


def _loss_kernel(costh_ref, label_ref, out_ref, se_acc, g_acc):
    jb = pl.program_id(0)

    @pl.when(jb == 0)
    def _init():
        se_acc[...] = jnp.zeros_like(se_acc)
        g_acc[...] = jnp.zeros_like(g_acc)

    lrel = label_ref[...] - jb * _W
    _H = _W // 2
    iota_h = lax.broadcasted_iota(jnp.int32, (_B, _H), 1)
    for h in range(2):
        ch = costh_ref[:, h * _H:(h + 1) * _H]
        is_lab = iota_h == (lrel - h * _H)
        g_acc[...] += jnp.sum(jnp.where(is_lab, ch, 0.0), axis=1, keepdims=True)

    @pl.when(jb < _NBLK - 1)
    def _main():
        for h in range(2):
            ch = costh_ref[:, h * _H:(h + 1) * _H]
            se_acc[...] += jnp.sum(jnp.exp(8.5 * ch), axis=1, keepdims=True)

    @pl.when(jb == _NBLK - 1)
    def _fin():
        e = jnp.where(
            lax.broadcasted_iota(jnp.int32, (_B, _W), 1) < (_C - jb * _W),
            jnp.exp(8.5 * costh_ref[...]), 0.0)
        se = se_acc[...] + jnp.sum(e, axis=1, keepdims=True)
        a_g = 8.5 * g_acc[...]
        picked = a_g - 2.25
        se = se - jnp.exp(a_g) + jnp.exp(picked)
        loss_i = jnp.log(se) - picked
        out_ref[...] = jnp.mean(loss_i, keepdims=True)


def kernel(costh, label):
    label2d = label.astype(jnp.int32).reshape(_B, 1)
    out = pl.pallas_call(
        _loss_kernel,
        grid=(_NBLK,),
        in_specs=[
            pl.BlockSpec((_B, _W), lambda j: (0, j)),
            pl.BlockSpec((_B, 1), lambda j: (0, 0)),
        ],
        out_specs=pl.BlockSpec((1, 1), lambda j: (0, 0)),
        out_shape=jax.ShapeDtypeStruct((1, 1), jnp.float32),
        scratch_shapes=[
            pltpu.VMEM((_B, 1), jnp.float32),
            pltpu.VMEM((_B, 1), jnp.float32),
        ],
    )(costh, label2d)
    return out[0, 0]


# speedup vs baseline: 1.0020x; 1.0020x over previous
import jax
import jax.numpy as jnp
from jax import lax
from jax.experimental import pallas as pl
from jax.experimental.pallas import tpu as pltpu

_B = 1024
_C = 100000
_W = 4096
_NBLK = (_C + _W - 1) // _W


def _loss_kernel(costh_ref, label_ref, out_ref, se_acc, g_acc):
    jb = pl.program_id(0)

    @pl.when(jb == 0)
    def _init():
        se_acc[...] = jnp.zeros_like(se_acc)
        g_acc[...] = jnp.zeros_like(g_acc)

    c = costh_ref[...]
    lrel = label_ref[...] - jb * _W
    is_lab = lax.broadcasted_iota(jnp.int32, (_B, _W), 1) == lrel
    g_acc[...] += jnp.sum(jnp.where(is_lab, c, 0.0), axis=1, keepdims=True)

    @pl.when(jb < _NBLK - 1)
    def _main():
        se_acc[...] += jnp.sum(jnp.exp(8.5 * c), axis=1, keepdims=True)

    @pl.when(jb == _NBLK - 1)
    def _fin():
        e = jnp.where(
            lax.broadcasted_iota(jnp.int32, (_B, _W), 1) < (_C - jb * _W),
            jnp.exp(8.5 * c), 0.0)
        se = se_acc[...] + jnp.sum(e, axis=1, keepdims=True)
        a_g = 8.5 * g_acc[...]
        picked = a_g - 2.25
        se = se - jnp.exp(a_g) + jnp.exp(picked)
        loss_i = jnp.log(se) - picked
        out_ref[...] = jnp.mean(loss_i, keepdims=True)


def kernel(costh, label):
    label2d = label.astype(jnp.int32).reshape(_B, 1)
    out = pl.pallas_call(
        _loss_kernel,
        grid=(_NBLK,),
        in_specs=[
            pl.BlockSpec((_B, _W), lambda j: (0, j)),
            pl.BlockSpec((_B, 1), lambda j: (0, 0)),
        ],
        out_specs=pl.BlockSpec((1, 1), lambda j: (0, 0)),
        out_shape=jax.ShapeDtypeStruct((1, 1), jnp.float32),
        scratch_shapes=[
            pltpu.VMEM((_B, 1), jnp.float32),
            pltpu.VMEM((_B, 1), jnp.float32),
        ],
    )(costh, label2d)
    return out[0, 0]
